# baseline (speedup 1.0000x reference)
import functools

import jax
import jax.numpy as jnp
from jax import lax
from jax.experimental import pallas as pl
from jax.experimental.pallas import tpu as pltpu
from jax.experimental.pallas import tpu_sc as plsc

N_NODES = 10000
N_EDGES = 320000
N_GRAPHS = 32
D = 128
NUM_RELS = 4
L_SIZE = 64
VOC = 38
H = 400
MAX_LEN = 151

_F32 = jnp.float32


def _dense0_body(x_ref, w_ref, ws_ref, b_ref, xw_ref, st_ref):
    h = x_ref[:]
    for r in range(NUM_RELS):
        xw_ref[pl.ds(r * N_NODES, N_NODES)] = jnp.dot(
            h, w_ref[r], preferred_element_type=_F32)
    st_ref[:] = jnp.dot(h, ws_ref[:], preferred_element_type=_F32) + b_ref[:]


def _densef_body(p_ref, sp_ref, w_ref, ws_ref, b_ref, xw_ref, st_ref):
    h = jnp.maximum(p_ref[0] + p_ref[1] + sp_ref[:], 0.0)
    for r in range(NUM_RELS):
        xw_ref[pl.ds(r * N_NODES, N_NODES)] = jnp.dot(
            h, w_ref[r], preferred_element_type=_F32)
    st_ref[:] = jnp.dot(h, ws_ref[:], preferred_element_type=_F32) + b_ref[:]


def _dense0(x, w, wself, b):
    return pl.pallas_call(
        _dense0_body,
        out_shape=(
            jax.ShapeDtypeStruct((NUM_RELS * N_NODES, D), _F32),
            jax.ShapeDtypeStruct((N_NODES, D), _F32),
        ),
    )(x, w, wself, b.reshape(1, D))


def _densef(parts, sterm, w, wself, b):
    return pl.pallas_call(
        _densef_body,
        out_shape=(
            jax.ShapeDtypeStruct((NUM_RELS * N_NODES, D), _F32),
            jax.ShapeDtypeStruct((N_NODES, D), _F32),
        ),
    )(parts, sterm, w, wself, b.reshape(1, D))


def _pool_body(p_ref, sp_ref, gid_ref, pooled_ref):
    h = jnp.maximum(p_ref[0] + p_ref[1] + sp_ref[:], 0.0)
    gid = gid_ref[:]
    seg = lax.broadcasted_iota(jnp.int32, (N_GRAPHS, 1), 0)
    mask = (gid == seg).astype(_F32)
    pooled_ref[:] = jnp.dot(mask, h, preferred_element_type=_F32)


def _pool(parts, sterm, gid):
    return pl.pallas_call(
        _pool_body,
        out_shape=jax.ShapeDtypeStruct((N_GRAPHS, D), _F32),
    )(parts, sterm, gid.reshape(1, N_NODES))


def _enc_body(pooled_ref, eps_ref, emw, emb, elw, elb, rnw, rnb, diw, dib,
              m0w, m0b, m1w, m1b, m2w, m2b, a0w, a0b, a1w, a1b, a2w, a2b,
              mu_ref, lv_ref, z_ref, r0_ref, d_ref, pr_ref, af_ref):
    p = pooled_ref[:]
    mu = jnp.dot(p, emw[:], preferred_element_type=_F32) + emb[:]
    lv = jnp.dot(p, elw[:], preferred_element_type=_F32) + elb[:]
    z = mu + eps_ref[:] * jnp.exp(0.5 * lv)
    mu_ref[:] = mu
    lv_ref[:] = lv
    z_ref[:] = z
    r0_ref[:] = jnp.dot(z, rnw[:], preferred_element_type=_F32) + rnb[:]
    d_ref[:] = jnp.dot(z, diw[:], preferred_element_type=_F32) + dib[:]
    q = jnp.maximum(jnp.dot(z, m0w[:], preferred_element_type=_F32) + m0b[:], 0.0)
    q = jnp.maximum(jnp.dot(q, m1w[:], preferred_element_type=_F32) + m1b[:], 0.0)
    pr_ref[:] = jnp.dot(q, m2w[:], preferred_element_type=_F32) + m2b[:]
    a = jnp.maximum(jnp.dot(z, a0w[:], preferred_element_type=_F32) + a0b[:], 0.0)
    a = jnp.maximum(jnp.dot(a, a1w[:], preferred_element_type=_F32) + a1b[:], 0.0)
    af_ref[:] = jax.nn.sigmoid(jnp.dot(a, a2w[:], preferred_element_type=_F32) + a2b[:])


def _encoder(pooled, eps2, pp):
    outs = pl.pallas_call(
        _enc_body,
        out_shape=(
            jax.ShapeDtypeStruct((N_GRAPHS, L_SIZE), _F32),
            jax.ShapeDtypeStruct((N_GRAPHS, L_SIZE), _F32),
            jax.ShapeDtypeStruct((N_GRAPHS, L_SIZE), _F32),
            jax.ShapeDtypeStruct((N_GRAPHS, VOC), _F32),
            jax.ShapeDtypeStruct((N_GRAPHS, 3 * H), _F32),
            jax.ShapeDtypeStruct((N_GRAPHS, 3), _F32),
            jax.ShapeDtypeStruct((N_GRAPHS, 2), _F32),
        ),
    )(
        pooled, eps2,
        pp['enc_mean_W'], pp['enc_mean_b'].reshape(1, -1),
        pp['enc_logv_W'], pp['enc_logv_b'].reshape(1, -1),
        pp['rnn_in_W'], pp['rnn_in_b'].reshape(1, -1),
        pp['dense_init_W'], pp['dense_init_b'].reshape(1, -1),
        pp['mlp0_W'], pp['mlp0_b'].reshape(1, -1),
        pp['mlp1_W'], pp['mlp1_b'].reshape(1, -1),
        pp['mlp2_W'], pp['mlp2_b'].reshape(1, -1),
        pp['aff0_W'], pp['aff0_b'].reshape(1, -1),
        pp['aff1_W'], pp['aff1_b'].reshape(1, -1),
        pp['aff2_W'], pp['aff2_b'].reshape(1, -1),
    )
    return outs


def _gru_body(sm_ref, r0_ref, h1_ref, h2_ref, h3_ref,
              w0i, w0h, b0i, b0h, w1i, w1h, b1i, b1h, w2i, w2h, b2i, b2h,
              lw, lb, out_ref):
    w0i_, w0h_ = w0i[:], w0h[:]
    w1i_, w1h_ = w1i[:], w1h[:]
    w2i_, w2h_ = w2i[:], w2h[:]
    b0i_, b0h_ = b0i[:], b0h[:]
    b1i_, b1h_ = b1i[:], b1h[:]
    b2i_, b2h_ = b2i[:], b2h[:]
    lw_, lb_ = lw[:], lb[:]

    bf = jnp.bfloat16

    def cell(xv, hv, wi, wh, bi, bh):
        gi = jnp.dot(xv.astype(bf), wi, preferred_element_type=_F32) + bi
        gh = jnp.dot(hv.astype(bf), wh, preferred_element_type=_F32) + bh
        r = jax.nn.sigmoid(gi[:, :H] + gh[:, :H])
        zt = jax.nn.sigmoid(gi[:, H:2 * H] + gh[:, H:2 * H])
        n = jnp.tanh(gi[:, 2 * H:] + r * gh[:, 2 * H:])
        return (1.0 - zt) * n + zt * hv

    def step(t, carry):
        xin, h1, h2, h3 = carry
        h1n = cell(xin, h1, w0i_, w0h_, b0i_, b0h_)
        h2n = cell(h1n, h2, w1i_, w1h_, b1i_, b1h_)
        h3n = cell(h2n, h3, w2i_, w2h_, b2i_, b2h_)
        out = jnp.dot(h3n.astype(bf), lw_, preferred_element_type=_F32) + lb_
        out_ref[pl.ds(t, 1)] = out[None]
        s = sm_ref[pl.ds(t, 1)]
        sc = s.reshape(N_GRAPHS, 1)
        oh = (sc == lax.broadcasted_iota(jnp.int32, (N_GRAPHS, VOC), 1))
        return (oh.astype(_F32), h1n, h2n, h3n)

    carry = (r0_ref[:], h1_ref[:], h2_ref[:], h3_ref[:])
    lax.fori_loop(0, MAX_LEN, step, carry)


def _gru(smiles, rnn0, hdec0, pp):
    sm = smiles.T.reshape(MAX_LEN, N_GRAPHS, 1)
    bf = jnp.bfloat16
    args = [sm, rnn0, hdec0[0], hdec0[1], hdec0[2]]
    for l in range(3):
        args.append(pp['gru%d_Wih' % l].T.astype(bf))
        args.append(pp['gru%d_Whh' % l].T.astype(bf))
        args.append(pp['gru%d_bih' % l].reshape(1, -1))
        args.append(pp['gru%d_bhh' % l].reshape(1, -1))
    args.append(pp['lin_W'].astype(bf))
    args.append(pp['lin_b'].reshape(1, -1))
    return pl.pallas_call(
        _gru_body,
        out_shape=jax.ShapeDtypeStruct((MAX_LEN, N_GRAPHS, VOC), _F32),
    )(*args)


_NC, _NS = 2, 16
_NW = _NC * _NS
_CH = 128
_NROWS = 2560
_AGG = 10240
_RPT = _AGG // _NS
_DUMMY = 10000
_ZR = 40
_SB = 16
_RPW = 80


def _sc_gather_scatter(table, fidx2d, dst2d):
    mesh = plsc.VectorSubcoreMesh(core_axis_name="c", subcore_axis_name="s")

    def _body(cid, sid, table_hbm, fidx_hbm, dst_hbm, out_hbm, fblk, dblk,
              rows_v, zbuf_v, agg_sh, sem_g, sem_s0, sem_s1, sem_i):
        zv = jnp.zeros((16,), _F32)

        def zstore(j, _):
            zbuf_v[j // 8, pl.ds((j % 8) * 16, 16)] = zv
            return 0

        lax.fori_loop(0, _ZR * 8, zstore, 0)
        for t in range(_RPT // _ZR):
            pltpu.sync_copy(zbuf_v, agg_sh.at[pl.ds(sid * _RPT + t * _ZR, _ZR)])
        plsc.subcore_barrier()

        base = (sid * _NC + cid) * _RPW
        nsb = _RPW // _SB
        pltpu.sync_copy(fidx_hbm.at[pl.ds(base, _SB)], fblk.at[0])
        pltpu.sync_copy(dst_hbm.at[pl.ds(base, _SB)], dblk.at[0])
        sem_s = (sem_s0, sem_s1)

        def superblock(s, _):
            sb = s % 2
            nb = (s + 1) % 2
            nxt = base + (s + 1) * _SB

            @pl.when(s < nsb - 1)
            def _pf():
                pltpu.async_copy(fidx_hbm.at[pl.ds(nxt, _SB)], fblk.at[nb], sem_i)
                pltpu.async_copy(dst_hbm.at[pl.ds(nxt, _SB)], dblk.at[nb], sem_i)

            pending = [None, None]
            for j in range(_SB):
                b = j % 2
                if pending[b] is not None:
                    pending[b].wait()
                pltpu.async_copy(table_hbm.at[fblk.at[sb, j]], rows_v.at[b],
                                 sem_g).wait()
                pending[b] = pltpu.async_copy(rows_v.at[b],
                                              agg_sh.at[dblk.at[sb, j]],
                                              sem_s[b], add=True)
            pending[0].wait()
            pending[1].wait()

            @pl.when(s < nsb - 1)
            def _pfw():
                pltpu.make_async_copy(fidx_hbm.at[pl.ds(nxt, _SB)], fblk.at[nb],
                                      sem_i).wait()
                pltpu.make_async_copy(dst_hbm.at[pl.ds(nxt, _SB)], dblk.at[nb],
                                      sem_i).wait()

            return 0

        lax.fori_loop(0, nsb, superblock, 0)

        plsc.subcore_barrier()
        rbase = sid * _RPT

        @pl.when(sid < _NS - 1)
        def _full():
            pltpu.sync_copy(agg_sh.at[pl.ds(rbase, _RPT)],
                            out_hbm.at[cid, pl.ds(rbase, _RPT)])

        @pl.when(sid == _NS - 1)
        def _last():
            lastbase = (_NS - 1) * _RPT
            pltpu.sync_copy(agg_sh.at[pl.ds(lastbase, N_NODES - lastbase)],
                            out_hbm.at[cid, pl.ds(lastbase, N_NODES - lastbase)])

    @functools.partial(
        pl.kernel, mesh=mesh,
        out_type=jax.ShapeDtypeStruct((_NC, N_NODES, D), _F32),
        scratch_types=[
            pltpu.VMEM((2, _SB, _CH), jnp.int32),
            pltpu.VMEM((2, _SB, _CH), jnp.int32),
            pltpu.VMEM((2, _CH, D), _F32),
            pltpu.VMEM((_ZR, D), _F32),
            pltpu.VMEM_SHARED((_AGG, D), _F32),
            pltpu.SemaphoreType.DMA,
            pltpu.SemaphoreType.DMA,
            pltpu.SemaphoreType.DMA,
            pltpu.SemaphoreType.DMA,
        ],
    )
    def k(table_hbm, fidx_hbm, dst_hbm, out_hbm, fblk, dblk, rows_v, zbuf_v,
          agg_sh, sem_g, sem_s0, sem_s1, sem_i):
        cid = lax.axis_index("c")
        sid = lax.axis_index("s")

        _body(cid, sid, table_hbm, fidx_hbm, dst_hbm, out_hbm, fblk, dblk,
              rows_v, zbuf_v, agg_sh, sem_g, sem_s0, sem_s1, sem_i)

    return k(table, fidx2d, dst2d)


def kernel(x, edge_index, etypes, graph_ids, smiles, params, eps):
    src = edge_index[0]
    dst = edge_index[1]
    npad = _NROWS * _CH - N_EDGES
    pad_ar = jnp.arange(npad, dtype=jnp.int32)
    fidx2d = jnp.concatenate(
        [etypes * N_NODES + src, pad_ar % (NUM_RELS * N_NODES)]).reshape(_NROWS, _CH)
    dst2d = jnp.concatenate(
        [dst, _DUMMY + pad_ar % (_AGG - _DUMMY)]).reshape(_NROWS, _CH)

    xw, sterm = _dense0(x, params['rgcn0_W'], params['rgcn0_Wself'], params['rgcn0_b'])
    parts = _sc_gather_scatter(xw, fidx2d, dst2d)
    for l in (1, 2, 3):
        xw, sterm = _densef(parts, sterm, params['rgcn%d_W' % l],
                            params['rgcn%d_Wself' % l], params['rgcn%d_b' % l])
        parts = _sc_gather_scatter(xw, fidx2d, dst2d)

    pooled = _pool(parts, sterm, graph_ids)
    mu2, lv2, z, rnn0, dinit, props, affs = _encoder(pooled, eps.reshape(N_GRAPHS, L_SIZE), params)
    hdec0 = dinit.reshape(3, N_GRAPHS, H)
    outs = _gru(smiles, rnn0, hdec0, params)
    gen_seq = jnp.transpose(outs, (1, 2, 0))
    mu = mu2.reshape(N_GRAPHS, 1, L_SIZE)
    logv = lv2.reshape(N_GRAPHS, 1, L_SIZE)
    return mu, logv, z, gen_seq, props, affs


# speedup vs baseline: 1.1228x; 1.1228x over previous
import functools

import jax
import jax.numpy as jnp
from jax import lax
from jax.experimental import pallas as pl
from jax.experimental.pallas import tpu as pltpu
from jax.experimental.pallas import tpu_sc as plsc

N_NODES = 10000
N_EDGES = 320000
N_GRAPHS = 32
D = 128
NUM_RELS = 4
L_SIZE = 64
VOC = 38
H = 400
MAX_LEN = 151

_F32 = jnp.float32


def _dense0_body(x_ref, w_ref, ws_ref, b_ref, xw_ref, st_ref):
    h = x_ref[:]
    for r in range(NUM_RELS):
        xw_ref[pl.ds(r * N_NODES, N_NODES)] = jnp.dot(
            h, w_ref[r], preferred_element_type=_F32)
    st_ref[:] = jnp.dot(h, ws_ref[:], preferred_element_type=_F32) + b_ref[:]


def _densef_body(p_ref, sp_ref, w_ref, ws_ref, b_ref, xw_ref, st_ref):
    h = jnp.maximum(p_ref[0] + p_ref[1] + sp_ref[:], 0.0)
    for r in range(NUM_RELS):
        xw_ref[pl.ds(r * N_NODES, N_NODES)] = jnp.dot(
            h, w_ref[r], preferred_element_type=_F32)
    st_ref[:] = jnp.dot(h, ws_ref[:], preferred_element_type=_F32) + b_ref[:]


def _dense0(x, w, wself, b):
    return pl.pallas_call(
        _dense0_body,
        out_shape=(
            jax.ShapeDtypeStruct((NUM_RELS * N_NODES, D), _F32),
            jax.ShapeDtypeStruct((N_NODES, D), _F32),
        ),
    )(x, w, wself, b.reshape(1, D))


def _densef(parts, sterm, w, wself, b):
    return pl.pallas_call(
        _densef_body,
        out_shape=(
            jax.ShapeDtypeStruct((NUM_RELS * N_NODES, D), _F32),
            jax.ShapeDtypeStruct((N_NODES, D), _F32),
        ),
    )(parts, sterm, w, wself, b.reshape(1, D))


def _pool_body(p_ref, sp_ref, gid_ref, pooled_ref):
    h = jnp.maximum(p_ref[0] + p_ref[1] + sp_ref[:], 0.0)
    gid = gid_ref[:]
    seg = lax.broadcasted_iota(jnp.int32, (N_GRAPHS, 1), 0)
    mask = (gid == seg).astype(_F32)
    pooled_ref[:] = jnp.dot(mask, h, preferred_element_type=_F32)


def _pool(parts, sterm, gid):
    return pl.pallas_call(
        _pool_body,
        out_shape=jax.ShapeDtypeStruct((N_GRAPHS, D), _F32),
    )(parts, sterm, gid.reshape(1, N_NODES))


def _enc_body(pooled_ref, eps_ref, emw, emb, elw, elb, rnw, rnb, diw, dib,
              m0w, m0b, m1w, m1b, m2w, m2b, a0w, a0b, a1w, a1b, a2w, a2b,
              mu_ref, lv_ref, z_ref, r0_ref, d_ref, pr_ref, af_ref):
    p = pooled_ref[:]
    mu = jnp.dot(p, emw[:], preferred_element_type=_F32) + emb[:]
    lv = jnp.dot(p, elw[:], preferred_element_type=_F32) + elb[:]
    z = mu + eps_ref[:] * jnp.exp(0.5 * lv)
    mu_ref[:] = mu
    lv_ref[:] = lv
    z_ref[:] = z
    r0_ref[:] = jnp.dot(z, rnw[:], preferred_element_type=_F32) + rnb[:]
    d_ref[:] = jnp.dot(z, diw[:], preferred_element_type=_F32) + dib[:]
    q = jnp.maximum(jnp.dot(z, m0w[:], preferred_element_type=_F32) + m0b[:], 0.0)
    q = jnp.maximum(jnp.dot(q, m1w[:], preferred_element_type=_F32) + m1b[:], 0.0)
    pr_ref[:] = jnp.dot(q, m2w[:], preferred_element_type=_F32) + m2b[:]
    a = jnp.maximum(jnp.dot(z, a0w[:], preferred_element_type=_F32) + a0b[:], 0.0)
    a = jnp.maximum(jnp.dot(a, a1w[:], preferred_element_type=_F32) + a1b[:], 0.0)
    af_ref[:] = jax.nn.sigmoid(jnp.dot(a, a2w[:], preferred_element_type=_F32) + a2b[:])


def _encoder(pooled, eps2, pp):
    outs = pl.pallas_call(
        _enc_body,
        out_shape=(
            jax.ShapeDtypeStruct((N_GRAPHS, L_SIZE), _F32),
            jax.ShapeDtypeStruct((N_GRAPHS, L_SIZE), _F32),
            jax.ShapeDtypeStruct((N_GRAPHS, L_SIZE), _F32),
            jax.ShapeDtypeStruct((N_GRAPHS, VOC), _F32),
            jax.ShapeDtypeStruct((N_GRAPHS, 3 * H), _F32),
            jax.ShapeDtypeStruct((N_GRAPHS, 3), _F32),
            jax.ShapeDtypeStruct((N_GRAPHS, 2), _F32),
        ),
    )(
        pooled, eps2,
        pp['enc_mean_W'], pp['enc_mean_b'].reshape(1, -1),
        pp['enc_logv_W'], pp['enc_logv_b'].reshape(1, -1),
        pp['rnn_in_W'], pp['rnn_in_b'].reshape(1, -1),
        pp['dense_init_W'], pp['dense_init_b'].reshape(1, -1),
        pp['mlp0_W'], pp['mlp0_b'].reshape(1, -1),
        pp['mlp1_W'], pp['mlp1_b'].reshape(1, -1),
        pp['mlp2_W'], pp['mlp2_b'].reshape(1, -1),
        pp['aff0_W'], pp['aff0_b'].reshape(1, -1),
        pp['aff1_W'], pp['aff1_b'].reshape(1, -1),
        pp['aff2_W'], pp['aff2_b'].reshape(1, -1),
    )
    return outs


def _gru_body(sm2_ref, r0_ref, h1_ref, h2_ref, h3_ref,
              w0i, w0h, b0i, b0h, w1i, w1h, b1i, b1h, w2i, w2h, b2i, b2h,
              lw, lb, out_ref, gi0t_ref, h3s_ref):
    bf = jnp.bfloat16
    w0h_, w1i_, w1h_, w2i_, w2h_ = w0h[:], w1i[:], w1h[:], w2i[:], w2h[:]
    b0i_, b0h_ = b0i[:], b0h[:]
    b1i_, b1h_ = b1i[:], b1h[:]
    b2i_, b2h_ = b2i[:], b2h[:]

    npre = (MAX_LEN - 1) * N_GRAPHS
    oh = (sm2_ref[:] == lax.broadcasted_iota(jnp.int32, (npre, VOC), 1)
          ).astype(bf)
    gi0t_ref[:] = jnp.dot(oh, w0i[:], preferred_element_type=_F32)

    def gates(gi, gh, hv):
        r = jax.nn.sigmoid(gi[:, :H] + gh[:, :H])
        zt = jax.nn.sigmoid(gi[:, H:2 * H] + gh[:, H:2 * H])
        n = jnp.tanh(gi[:, 2 * H:] + r * gh[:, 2 * H:])
        return (1.0 - zt) * n + zt * hv

    def cells(gi0, h1, h2, h3):
        gh1 = jnp.dot(h1.astype(bf), w0h_, preferred_element_type=_F32) + b0h_
        gh2 = jnp.dot(h2.astype(bf), w1h_, preferred_element_type=_F32) + b1h_
        gh3 = jnp.dot(h3.astype(bf), w2h_, preferred_element_type=_F32) + b2h_
        h1n = gates(gi0, gh1, h1)
        gi1 = jnp.dot(h1n.astype(bf), w1i_, preferred_element_type=_F32) + b1i_
        h2n = gates(gi1, gh2, h2)
        gi2 = jnp.dot(h2n.astype(bf), w2i_, preferred_element_type=_F32) + b2i_
        h3n = gates(gi2, gh3, h3)
        return h1n, h2n, h3n

    gi0 = jnp.dot(r0_ref[:].astype(bf), w0i[:], preferred_element_type=_F32) + b0i_
    h1, h2, h3 = cells(gi0, h1_ref[:], h2_ref[:], h3_ref[:])
    h3s_ref[pl.ds(0, 1)] = h3[None]

    def step(t, carry):
        c1, c2, c3 = carry
        g = gi0t_ref[pl.ds((t - 1) * N_GRAPHS, N_GRAPHS)] + b0i_
        c1, c2, c3 = cells(g, c1, c2, c3)
        h3s_ref[pl.ds(t, 1)] = c3[None]
        return (c1, c2, c3)

    lax.fori_loop(1, MAX_LEN, step, (h1, h2, h3))
    h3all = h3s_ref[:].reshape(MAX_LEN * N_GRAPHS, H)
    out_ref[:] = jnp.dot(h3all.astype(bf), lw[:], preferred_element_type=_F32) + lb[:]


def _gru(smiles, rnn0, hdec0, pp):
    sm2 = smiles[:, :MAX_LEN - 1].T.reshape((MAX_LEN - 1) * N_GRAPHS, 1)
    bf = jnp.bfloat16
    args = [sm2, rnn0, hdec0[0], hdec0[1], hdec0[2]]
    for l in range(3):
        args.append(pp['gru%d_Wih' % l].T.astype(bf))
        args.append(pp['gru%d_Whh' % l].T.astype(bf))
        args.append(pp['gru%d_bih' % l].reshape(1, -1))
        args.append(pp['gru%d_bhh' % l].reshape(1, -1))
    args.append(pp['lin_W'].astype(bf))
    args.append(pp['lin_b'].reshape(1, -1))
    return pl.pallas_call(
        _gru_body,
        out_shape=jax.ShapeDtypeStruct((MAX_LEN * N_GRAPHS, VOC), _F32),
        scratch_shapes=[
            pltpu.VMEM(((MAX_LEN - 1) * N_GRAPHS, 3 * H), _F32),
            pltpu.VMEM((MAX_LEN, N_GRAPHS, H), _F32),
        ],
    )(*args)


_NC, _NS = 2, 16
_NW = _NC * _NS
_CH = 128
_NROWS = 2560
_AGG = 10240
_RPT = _AGG // _NS
_DUMMY = 10000
_ZR = 40
_SB = 16
_RPW = 80


def _sc_gather_scatter(table, fidx2d, dst2d):
    mesh = plsc.VectorSubcoreMesh(core_axis_name="c", subcore_axis_name="s")

    def _body(cid, sid, table_hbm, fidx_hbm, dst_hbm, out_hbm, fblk, dblk,
              rows_v, zbuf_v, agg_sh, sem_g, sem_s0, sem_s1, sem_i):
        zv = jnp.zeros((16,), _F32)

        def zstore(j, _):
            zbuf_v[j // 8, pl.ds((j % 8) * 16, 16)] = zv
            return 0

        lax.fori_loop(0, _ZR * 8, zstore, 0)
        for t in range(_RPT // _ZR):
            pltpu.sync_copy(zbuf_v, agg_sh.at[pl.ds(sid * _RPT + t * _ZR, _ZR)])
        plsc.subcore_barrier()

        base = (sid * _NC + cid) * _RPW
        nsb = _RPW // _SB
        pltpu.sync_copy(fidx_hbm.at[pl.ds(base, _SB)], fblk.at[0])
        pltpu.sync_copy(dst_hbm.at[pl.ds(base, _SB)], dblk.at[0])
        sem_s = (sem_s0, sem_s1)

        def superblock(s, _):
            sb = s % 2
            nb = (s + 1) % 2
            nxt = base + (s + 1) * _SB

            @pl.when(s < nsb - 1)
            def _pf():
                pltpu.async_copy(fidx_hbm.at[pl.ds(nxt, _SB)], fblk.at[nb], sem_i)
                pltpu.async_copy(dst_hbm.at[pl.ds(nxt, _SB)], dblk.at[nb], sem_i)

            pending = [None, None]
            for j in range(_SB):
                b = j % 2
                if pending[b] is not None:
                    pending[b].wait()
                pltpu.async_copy(table_hbm.at[fblk.at[sb, j]], rows_v.at[b],
                                 sem_g).wait()
                pending[b] = pltpu.async_copy(rows_v.at[b],
                                              agg_sh.at[dblk.at[sb, j]],
                                              sem_s[b], add=True)
            pending[0].wait()
            pending[1].wait()

            @pl.when(s < nsb - 1)
            def _pfw():
                pltpu.make_async_copy(fidx_hbm.at[pl.ds(nxt, _SB)], fblk.at[nb],
                                      sem_i).wait()
                pltpu.make_async_copy(dst_hbm.at[pl.ds(nxt, _SB)], dblk.at[nb],
                                      sem_i).wait()

            return 0

        lax.fori_loop(0, nsb, superblock, 0)

        plsc.subcore_barrier()
        rbase = sid * _RPT

        @pl.when(sid < _NS - 1)
        def _full():
            pltpu.sync_copy(agg_sh.at[pl.ds(rbase, _RPT)],
                            out_hbm.at[cid, pl.ds(rbase, _RPT)])

        @pl.when(sid == _NS - 1)
        def _last():
            lastbase = (_NS - 1) * _RPT
            pltpu.sync_copy(agg_sh.at[pl.ds(lastbase, N_NODES - lastbase)],
                            out_hbm.at[cid, pl.ds(lastbase, N_NODES - lastbase)])

    @functools.partial(
        pl.kernel, mesh=mesh,
        out_type=jax.ShapeDtypeStruct((_NC, N_NODES, D), _F32),
        scratch_types=[
            pltpu.VMEM((2, _SB, _CH), jnp.int32),
            pltpu.VMEM((2, _SB, _CH), jnp.int32),
            pltpu.VMEM((2, _CH, D), _F32),
            pltpu.VMEM((_ZR, D), _F32),
            pltpu.VMEM_SHARED((_AGG, D), _F32),
            pltpu.SemaphoreType.DMA,
            pltpu.SemaphoreType.DMA,
            pltpu.SemaphoreType.DMA,
            pltpu.SemaphoreType.DMA,
        ],
    )
    def k(table_hbm, fidx_hbm, dst_hbm, out_hbm, fblk, dblk, rows_v, zbuf_v,
          agg_sh, sem_g, sem_s0, sem_s1, sem_i):
        cid = lax.axis_index("c")
        sid = lax.axis_index("s")

        _body(cid, sid, table_hbm, fidx_hbm, dst_hbm, out_hbm, fblk, dblk,
              rows_v, zbuf_v, agg_sh, sem_g, sem_s0, sem_s1, sem_i)

    return k(table, fidx2d, dst2d)


def kernel(x, edge_index, etypes, graph_ids, smiles, params, eps):
    src = edge_index[0]
    dst = edge_index[1]
    npad = _NROWS * _CH - N_EDGES
    pad_ar = jnp.arange(npad, dtype=jnp.int32)
    fidx2d = jnp.concatenate(
        [etypes * N_NODES + src, pad_ar % (NUM_RELS * N_NODES)]).reshape(_NROWS, _CH)
    dst2d = jnp.concatenate(
        [dst, _DUMMY + pad_ar % (_AGG - _DUMMY)]).reshape(_NROWS, _CH)

    xw, sterm = _dense0(x, params['rgcn0_W'], params['rgcn0_Wself'], params['rgcn0_b'])
    parts = _sc_gather_scatter(xw, fidx2d, dst2d)
    for l in (1, 2, 3):
        xw, sterm = _densef(parts, sterm, params['rgcn%d_W' % l],
                            params['rgcn%d_Wself' % l], params['rgcn%d_b' % l])
        parts = _sc_gather_scatter(xw, fidx2d, dst2d)

    pooled = _pool(parts, sterm, graph_ids)
    mu2, lv2, z, rnn0, dinit, props, affs = _encoder(pooled, eps.reshape(N_GRAPHS, L_SIZE), params)
    hdec0 = dinit.reshape(3, N_GRAPHS, H)
    outs = _gru(smiles, rnn0, hdec0, params).reshape(MAX_LEN, N_GRAPHS, VOC)
    gen_seq = jnp.transpose(outs, (1, 2, 0))
    mu = mu2.reshape(N_GRAPHS, 1, L_SIZE)
    logv = lv2.reshape(N_GRAPHS, 1, L_SIZE)
    return mu, logv, z, gen_seq, props, affs


# speedup vs baseline: 1.1419x; 1.0169x over previous
import functools

import jax
import jax.numpy as jnp
from jax import lax
from jax.experimental import pallas as pl
from jax.experimental.pallas import tpu as pltpu
from jax.experimental.pallas import tpu_sc as plsc

N_NODES = 10000
N_EDGES = 320000
N_GRAPHS = 32
D = 128
NUM_RELS = 4
L_SIZE = 64
VOC = 38
H = 400
MAX_LEN = 151

_F32 = jnp.float32


def _dense0_body(x_ref, w_ref, ws_ref, b_ref, xw_ref, st_ref):
    h = x_ref[:]
    for r in range(NUM_RELS):
        xw_ref[pl.ds(r * N_NODES, N_NODES)] = jnp.dot(
            h, w_ref[r], preferred_element_type=_F32)
    st_ref[:] = jnp.dot(h, ws_ref[:], preferred_element_type=_F32) + b_ref[:]


def _densef_body(p_ref, sp_ref, w_ref, ws_ref, b_ref, xw_ref, st_ref):
    h = jnp.maximum(p_ref[0] + p_ref[1] + sp_ref[:], 0.0)
    for r in range(NUM_RELS):
        xw_ref[pl.ds(r * N_NODES, N_NODES)] = jnp.dot(
            h, w_ref[r], preferred_element_type=_F32)
    st_ref[:] = jnp.dot(h, ws_ref[:], preferred_element_type=_F32) + b_ref[:]


def _dense0(x, w, wself, b):
    return pl.pallas_call(
        _dense0_body,
        out_shape=(
            jax.ShapeDtypeStruct((NUM_RELS * N_NODES, D), _F32),
            jax.ShapeDtypeStruct((N_NODES, D), _F32),
        ),
    )(x, w, wself, b.reshape(1, D))


def _densef(parts, sterm, w, wself, b):
    return pl.pallas_call(
        _densef_body,
        out_shape=(
            jax.ShapeDtypeStruct((NUM_RELS * N_NODES, D), _F32),
            jax.ShapeDtypeStruct((N_NODES, D), _F32),
        ),
    )(parts, sterm, w, wself, b.reshape(1, D))


def _pool_body(p_ref, sp_ref, gid_ref, pooled_ref):
    h = jnp.maximum(p_ref[0] + p_ref[1] + sp_ref[:], 0.0)
    gid = gid_ref[:]
    seg = lax.broadcasted_iota(jnp.int32, (N_GRAPHS, 1), 0)
    mask = (gid == seg).astype(_F32)
    pooled_ref[:] = jnp.dot(mask, h, preferred_element_type=_F32)


def _pool(parts, sterm, gid):
    return pl.pallas_call(
        _pool_body,
        out_shape=jax.ShapeDtypeStruct((N_GRAPHS, D), _F32),
    )(parts, sterm, gid.reshape(1, N_NODES))


def _enc_body(pooled_ref, eps_ref, emw, emb, elw, elb, rnw, rnb, diw, dib,
              m0w, m0b, m1w, m1b, m2w, m2b, a0w, a0b, a1w, a1b, a2w, a2b,
              mu_ref, lv_ref, z_ref, r0_ref, d_ref, pr_ref, af_ref):
    p = pooled_ref[:]
    mu = jnp.dot(p, emw[:], preferred_element_type=_F32) + emb[:]
    lv = jnp.dot(p, elw[:], preferred_element_type=_F32) + elb[:]
    z = mu + eps_ref[:] * jnp.exp(0.5 * lv)
    mu_ref[:] = mu
    lv_ref[:] = lv
    z_ref[:] = z
    r0_ref[:] = jnp.dot(z, rnw[:], preferred_element_type=_F32) + rnb[:]
    d_ref[:] = jnp.dot(z, diw[:], preferred_element_type=_F32) + dib[:]
    q = jnp.maximum(jnp.dot(z, m0w[:], preferred_element_type=_F32) + m0b[:], 0.0)
    q = jnp.maximum(jnp.dot(q, m1w[:], preferred_element_type=_F32) + m1b[:], 0.0)
    pr_ref[:] = jnp.dot(q, m2w[:], preferred_element_type=_F32) + m2b[:]
    a = jnp.maximum(jnp.dot(z, a0w[:], preferred_element_type=_F32) + a0b[:], 0.0)
    a = jnp.maximum(jnp.dot(a, a1w[:], preferred_element_type=_F32) + a1b[:], 0.0)
    af_ref[:] = jax.nn.sigmoid(jnp.dot(a, a2w[:], preferred_element_type=_F32) + a2b[:])


def _encoder(pooled, eps2, pp):
    outs = pl.pallas_call(
        _enc_body,
        out_shape=(
            jax.ShapeDtypeStruct((N_GRAPHS, L_SIZE), _F32),
            jax.ShapeDtypeStruct((N_GRAPHS, L_SIZE), _F32),
            jax.ShapeDtypeStruct((N_GRAPHS, L_SIZE), _F32),
            jax.ShapeDtypeStruct((N_GRAPHS, VOC), _F32),
            jax.ShapeDtypeStruct((N_GRAPHS, 3 * H), _F32),
            jax.ShapeDtypeStruct((N_GRAPHS, 3), _F32),
            jax.ShapeDtypeStruct((N_GRAPHS, 2), _F32),
        ),
    )(
        pooled, eps2,
        pp['enc_mean_W'], pp['enc_mean_b'].reshape(1, -1),
        pp['enc_logv_W'], pp['enc_logv_b'].reshape(1, -1),
        pp['rnn_in_W'], pp['rnn_in_b'].reshape(1, -1),
        pp['dense_init_W'], pp['dense_init_b'].reshape(1, -1),
        pp['mlp0_W'], pp['mlp0_b'].reshape(1, -1),
        pp['mlp1_W'], pp['mlp1_b'].reshape(1, -1),
        pp['mlp2_W'], pp['mlp2_b'].reshape(1, -1),
        pp['aff0_W'], pp['aff0_b'].reshape(1, -1),
        pp['aff1_W'], pp['aff1_b'].reshape(1, -1),
        pp['aff2_W'], pp['aff2_b'].reshape(1, -1),
    )
    return outs


def _gru_body(sm2_ref, r0_ref, h1_ref, h2_ref, h3_ref,
              w0i, w0h, b0i, b0h, w1i, w1h, b1i, b1h, w2i, w2h, b2i, b2h,
              lw, lb, out_ref, gi0t_ref, h3s_ref):
    bf = jnp.bfloat16
    w0h_, w1i_, w1h_, w2i_, w2h_ = w0h[:], w1i[:], w1h[:], w2i[:], w2h[:]
    b0i_, b0h_ = b0i[:], b0h[:]
    b1i_, b1h_ = b1i[:], b1h[:]
    b2i_, b2h_ = b2i[:], b2h[:]

    npre = (MAX_LEN - 1) * N_GRAPHS
    oh = (sm2_ref[:] == lax.broadcasted_iota(jnp.int32, (npre, VOC), 1)
          ).astype(bf)
    gi0t_ref[:] = jnp.dot(oh, w0i[:], preferred_element_type=_F32)

    def gates(gi, gh, hv):
        r = jax.nn.sigmoid(gi[:, :H] + gh[:, :H])
        zt = jax.nn.sigmoid(gi[:, H:2 * H] + gh[:, H:2 * H])
        n = jnp.tanh(gi[:, 2 * H:] + r * gh[:, 2 * H:])
        return (1.0 - zt) * n + zt * hv

    def cells(gi0, h1, h2, h3):
        gh1 = jnp.dot(h1.astype(bf), w0h_, preferred_element_type=_F32) + b0h_
        gh2 = jnp.dot(h2.astype(bf), w1h_, preferred_element_type=_F32) + b1h_
        gh3 = jnp.dot(h3.astype(bf), w2h_, preferred_element_type=_F32) + b2h_
        h1n = gates(gi0, gh1, h1)
        gi1 = jnp.dot(h1n.astype(bf), w1i_, preferred_element_type=_F32) + b1i_
        h2n = gates(gi1, gh2, h2)
        gi2 = jnp.dot(h2n.astype(bf), w2i_, preferred_element_type=_F32) + b2i_
        h3n = gates(gi2, gh3, h3)
        return h1n, h2n, h3n

    gi0 = jnp.dot(r0_ref[:].astype(bf), w0i[:], preferred_element_type=_F32) + b0i_
    h1, h2, h3 = cells(gi0, h1_ref[:], h2_ref[:], h3_ref[:])
    h3s_ref[pl.ds(0, 1)] = h3[None]

    def substep(t, carry):
        c1, c2, c3 = carry
        g = gi0t_ref[pl.ds((t - 1) * N_GRAPHS, N_GRAPHS)] + b0i_
        c1, c2, c3 = cells(g, c1, c2, c3)
        h3s_ref[pl.ds(t, 1)] = c3[None]
        return (c1, c2, c3)

    def step2(k, carry):
        carry = substep(2 * k + 1, carry)
        return substep(2 * k + 2, carry)

    lax.fori_loop(0, (MAX_LEN - 1) // 2, step2, (h1, h2, h3))
    h3all = h3s_ref[:].reshape(MAX_LEN * N_GRAPHS, H)
    out_ref[:] = jnp.dot(h3all.astype(bf), lw[:], preferred_element_type=_F32) + lb[:]


def _gru(smiles, rnn0, hdec0, pp):
    sm2 = smiles[:, :MAX_LEN - 1].T.reshape((MAX_LEN - 1) * N_GRAPHS, 1)
    bf = jnp.bfloat16
    args = [sm2, rnn0, hdec0[0], hdec0[1], hdec0[2]]
    for l in range(3):
        args.append(pp['gru%d_Wih' % l].T.astype(bf))
        args.append(pp['gru%d_Whh' % l].T.astype(bf))
        args.append(pp['gru%d_bih' % l].reshape(1, -1))
        args.append(pp['gru%d_bhh' % l].reshape(1, -1))
    args.append(pp['lin_W'].astype(bf))
    args.append(pp['lin_b'].reshape(1, -1))
    return pl.pallas_call(
        _gru_body,
        out_shape=jax.ShapeDtypeStruct((MAX_LEN * N_GRAPHS, VOC), _F32),
        scratch_shapes=[
            pltpu.VMEM(((MAX_LEN - 1) * N_GRAPHS, 3 * H), _F32),
            pltpu.VMEM((MAX_LEN, N_GRAPHS, H), _F32),
        ],
    )(*args)


_NC, _NS = 2, 16
_NW = _NC * _NS
_CH = 128
_NROWS = 2560
_AGG = 10240
_RPT = _AGG // _NS
_DUMMY = 10000
_ZR = 40
_SB = 16
_RPW = 80


def _sc_gather_scatter(table, fidx2d, dst2d):
    mesh = plsc.VectorSubcoreMesh(core_axis_name="c", subcore_axis_name="s")

    def _body(cid, sid, table_hbm, fidx_hbm, dst_hbm, out_hbm, fblk, dblk,
              rows_v, zbuf_v, agg_sh, sem_g, sem_s0, sem_s1, sem_i):
        zv = jnp.zeros((16,), _F32)

        def zstore(j, _):
            zbuf_v[j // 8, pl.ds((j % 8) * 16, 16)] = zv
            return 0

        lax.fori_loop(0, _ZR * 8, zstore, 0)
        for t in range(_RPT // _ZR):
            pltpu.sync_copy(zbuf_v, agg_sh.at[pl.ds(sid * _RPT + t * _ZR, _ZR)])
        plsc.subcore_barrier()

        base = (sid * _NC + cid) * _RPW
        nsb = _RPW // _SB
        pltpu.sync_copy(fidx_hbm.at[pl.ds(base, _SB)], fblk.at[0])
        pltpu.sync_copy(dst_hbm.at[pl.ds(base, _SB)], dblk.at[0])
        sem_s = (sem_s0, sem_s1)

        def superblock(s, _):
            sb = s % 2
            nb = (s + 1) % 2
            nxt = base + (s + 1) * _SB

            @pl.when(s < nsb - 1)
            def _pf():
                pltpu.async_copy(fidx_hbm.at[pl.ds(nxt, _SB)], fblk.at[nb], sem_i)
                pltpu.async_copy(dst_hbm.at[pl.ds(nxt, _SB)], dblk.at[nb], sem_i)

            pending = [None, None]
            for j in range(_SB):
                b = j % 2
                if pending[b] is not None:
                    pending[b].wait()
                pltpu.async_copy(table_hbm.at[fblk.at[sb, j]], rows_v.at[b],
                                 sem_g).wait()
                pending[b] = pltpu.async_copy(rows_v.at[b],
                                              agg_sh.at[dblk.at[sb, j]],
                                              sem_s[b], add=True)
            pending[0].wait()
            pending[1].wait()

            @pl.when(s < nsb - 1)
            def _pfw():
                pltpu.make_async_copy(fidx_hbm.at[pl.ds(nxt, _SB)], fblk.at[nb],
                                      sem_i).wait()
                pltpu.make_async_copy(dst_hbm.at[pl.ds(nxt, _SB)], dblk.at[nb],
                                      sem_i).wait()

            return 0

        lax.fori_loop(0, nsb, superblock, 0)

        plsc.subcore_barrier()
        rbase = sid * _RPT

        @pl.when(sid < _NS - 1)
        def _full():
            pltpu.sync_copy(agg_sh.at[pl.ds(rbase, _RPT)],
                            out_hbm.at[cid, pl.ds(rbase, _RPT)])

        @pl.when(sid == _NS - 1)
        def _last():
            lastbase = (_NS - 1) * _RPT
            pltpu.sync_copy(agg_sh.at[pl.ds(lastbase, N_NODES - lastbase)],
                            out_hbm.at[cid, pl.ds(lastbase, N_NODES - lastbase)])

    @functools.partial(
        pl.kernel, mesh=mesh,
        out_type=jax.ShapeDtypeStruct((_NC, N_NODES, D), _F32),
        scratch_types=[
            pltpu.VMEM((2, _SB, _CH), jnp.int32),
            pltpu.VMEM((2, _SB, _CH), jnp.int32),
            pltpu.VMEM((2, _CH, D), _F32),
            pltpu.VMEM((_ZR, D), _F32),
            pltpu.VMEM_SHARED((_AGG, D), _F32),
            pltpu.SemaphoreType.DMA,
            pltpu.SemaphoreType.DMA,
            pltpu.SemaphoreType.DMA,
            pltpu.SemaphoreType.DMA,
        ],
    )
    def k(table_hbm, fidx_hbm, dst_hbm, out_hbm, fblk, dblk, rows_v, zbuf_v,
          agg_sh, sem_g, sem_s0, sem_s1, sem_i):
        cid = lax.axis_index("c")
        sid = lax.axis_index("s")

        _body(cid, sid, table_hbm, fidx_hbm, dst_hbm, out_hbm, fblk, dblk,
              rows_v, zbuf_v, agg_sh, sem_g, sem_s0, sem_s1, sem_i)

    return k(table, fidx2d, dst2d)


def _build_idx_body(et_ref, src_ref, dst_ref, f_ref, d_ref):
    nreal = N_EDGES // _CH
    npadr = _NROWS - nreal
    f_ref[pl.ds(0, nreal)] = et_ref[:] * N_NODES + src_ref[:]
    d_ref[pl.ds(0, nreal)] = dst_ref[:]
    flat = (lax.broadcasted_iota(jnp.int32, (npadr, _CH), 0) * _CH
            + lax.broadcasted_iota(jnp.int32, (npadr, _CH), 1))
    f_ref[pl.ds(nreal, npadr)] = flat % (NUM_RELS * N_NODES)
    d_ref[pl.ds(nreal, npadr)] = _DUMMY + flat % (_AGG - _DUMMY)


def _build_idx(et2d, src2d, dst2d):
    return pl.pallas_call(
        _build_idx_body,
        out_shape=(
            jax.ShapeDtypeStruct((_NROWS, _CH), jnp.int32),
            jax.ShapeDtypeStruct((_NROWS, _CH), jnp.int32),
        ),
    )(et2d, src2d, dst2d)


def kernel(x, edge_index, etypes, graph_ids, smiles, params, eps):
    src = edge_index[0]
    dst = edge_index[1]
    fidx2d, dst2d = _build_idx(etypes.reshape(-1, _CH), src.reshape(-1, _CH),
                               dst.reshape(-1, _CH))

    xw, sterm = _dense0(x, params['rgcn0_W'], params['rgcn0_Wself'], params['rgcn0_b'])
    parts = _sc_gather_scatter(xw, fidx2d, dst2d)
    for l in (1, 2, 3):
        xw, sterm = _densef(parts, sterm, params['rgcn%d_W' % l],
                            params['rgcn%d_Wself' % l], params['rgcn%d_b' % l])
        parts = _sc_gather_scatter(xw, fidx2d, dst2d)

    pooled = _pool(parts, sterm, graph_ids)
    mu2, lv2, z, rnn0, dinit, props, affs = _encoder(pooled, eps.reshape(N_GRAPHS, L_SIZE), params)
    hdec0 = dinit.reshape(3, N_GRAPHS, H)
    outs = _gru(smiles, rnn0, hdec0, params).reshape(MAX_LEN, N_GRAPHS, VOC)
    gen_seq = jnp.transpose(outs, (1, 2, 0))
    mu = mu2.reshape(N_GRAPHS, 1, L_SIZE)
    logv = lv2.reshape(N_GRAPHS, 1, L_SIZE)
    return mu, logv, z, gen_seq, props, affs


# speedup vs baseline: 1.2502x; 1.0949x over previous
import functools

import jax
import jax.numpy as jnp
from jax import lax
from jax.experimental import pallas as pl
from jax.experimental.pallas import tpu as pltpu
from jax.experimental.pallas import tpu_sc as plsc

N_NODES = 10000
N_EDGES = 320000
N_GRAPHS = 32
D = 128
NUM_RELS = 4
L_SIZE = 64
VOC = 38
H = 400
MAX_LEN = 151

_F32 = jnp.float32


def _dense0_body(x_ref, w_ref, ws_ref, b_ref, xw_ref, st_ref):
    h = x_ref[:]
    for r in range(NUM_RELS):
        xw_ref[pl.ds(r * N_NODES, N_NODES)] = jnp.dot(
            h, w_ref[r], preferred_element_type=_F32)
    st_ref[:] = jnp.dot(h, ws_ref[:], preferred_element_type=_F32) + b_ref[:]


def _densef_body(p_ref, sp_ref, w_ref, ws_ref, b_ref, xw_ref, st_ref):
    h = jnp.maximum(p_ref[0] + p_ref[1] + sp_ref[:], 0.0)
    for r in range(NUM_RELS):
        xw_ref[pl.ds(r * N_NODES, N_NODES)] = jnp.dot(
            h, w_ref[r], preferred_element_type=_F32)
    st_ref[:] = jnp.dot(h, ws_ref[:], preferred_element_type=_F32) + b_ref[:]


def _dense0(x, w, wself, b):
    return pl.pallas_call(
        _dense0_body,
        out_shape=(
            jax.ShapeDtypeStruct((NUM_RELS * N_NODES, D), _F32),
            jax.ShapeDtypeStruct((N_NODES, D), _F32),
        ),
    )(x, w, wself, b.reshape(1, D))


def _densef(parts, sterm, w, wself, b):
    return pl.pallas_call(
        _densef_body,
        out_shape=(
            jax.ShapeDtypeStruct((NUM_RELS * N_NODES, D), _F32),
            jax.ShapeDtypeStruct((N_NODES, D), _F32),
        ),
    )(parts, sterm, w, wself, b.reshape(1, D))


def _pool_body(p_ref, sp_ref, gid_ref, pooled_ref):
    h = jnp.maximum(p_ref[0] + p_ref[1] + sp_ref[:], 0.0)
    gid = gid_ref[:]
    seg = lax.broadcasted_iota(jnp.int32, (N_GRAPHS, 1), 0)
    mask = (gid == seg).astype(_F32)
    pooled_ref[:] = jnp.dot(mask, h, preferred_element_type=_F32)


def _pool(parts, sterm, gid):
    return pl.pallas_call(
        _pool_body,
        out_shape=jax.ShapeDtypeStruct((N_GRAPHS, D), _F32),
    )(parts, sterm, gid.reshape(1, N_NODES))


def _enc_body(pooled_ref, eps_ref, emw, emb, elw, elb, rnw, rnb, diw, dib,
              m0w, m0b, m1w, m1b, m2w, m2b, a0w, a0b, a1w, a1b, a2w, a2b,
              mu_ref, lv_ref, z_ref, r0_ref, d_ref, pr_ref, af_ref):
    p = pooled_ref[:]
    mu = jnp.dot(p, emw[:], preferred_element_type=_F32) + emb[:]
    lv = jnp.dot(p, elw[:], preferred_element_type=_F32) + elb[:]
    z = mu + eps_ref[:] * jnp.exp(0.5 * lv)
    mu_ref[:] = mu
    lv_ref[:] = lv
    z_ref[:] = z
    r0_ref[:] = jnp.dot(z, rnw[:], preferred_element_type=_F32) + rnb[:]
    d_ref[:] = jnp.dot(z, diw[:], preferred_element_type=_F32) + dib[:]
    q = jnp.maximum(jnp.dot(z, m0w[:], preferred_element_type=_F32) + m0b[:], 0.0)
    q = jnp.maximum(jnp.dot(q, m1w[:], preferred_element_type=_F32) + m1b[:], 0.0)
    pr_ref[:] = jnp.dot(q, m2w[:], preferred_element_type=_F32) + m2b[:]
    a = jnp.maximum(jnp.dot(z, a0w[:], preferred_element_type=_F32) + a0b[:], 0.0)
    a = jnp.maximum(jnp.dot(a, a1w[:], preferred_element_type=_F32) + a1b[:], 0.0)
    af_ref[:] = jax.nn.sigmoid(jnp.dot(a, a2w[:], preferred_element_type=_F32) + a2b[:])


def _encoder(pooled, eps2, pp):
    outs = pl.pallas_call(
        _enc_body,
        out_shape=(
            jax.ShapeDtypeStruct((N_GRAPHS, L_SIZE), _F32),
            jax.ShapeDtypeStruct((N_GRAPHS, L_SIZE), _F32),
            jax.ShapeDtypeStruct((N_GRAPHS, L_SIZE), _F32),
            jax.ShapeDtypeStruct((N_GRAPHS, VOC), _F32),
            jax.ShapeDtypeStruct((N_GRAPHS, 3 * H), _F32),
            jax.ShapeDtypeStruct((N_GRAPHS, 3), _F32),
            jax.ShapeDtypeStruct((N_GRAPHS, 2), _F32),
        ),
    )(
        pooled, eps2,
        pp['enc_mean_W'], pp['enc_mean_b'].reshape(1, -1),
        pp['enc_logv_W'], pp['enc_logv_b'].reshape(1, -1),
        pp['rnn_in_W'], pp['rnn_in_b'].reshape(1, -1),
        pp['dense_init_W'], pp['dense_init_b'].reshape(1, -1),
        pp['mlp0_W'], pp['mlp0_b'].reshape(1, -1),
        pp['mlp1_W'], pp['mlp1_b'].reshape(1, -1),
        pp['mlp2_W'], pp['mlp2_b'].reshape(1, -1),
        pp['aff0_W'], pp['aff0_b'].reshape(1, -1),
        pp['aff1_W'], pp['aff1_b'].reshape(1, -1),
        pp['aff2_W'], pp['aff2_b'].reshape(1, -1),
    )
    return outs


def _gru_body(sm2_ref, r0_ref, h1_ref, h2_ref, h3_ref,
              w0i, w0h, b0i, b0h, w1i, w1h, b1i, b1h, w2i, w2h, b2i, b2h,
              lw, lb, out_ref, gi0t_ref, h3s_ref):
    bf = jnp.bfloat16
    w0h_, w1i_, w1h_, w2i_, w2h_ = w0h[:], w1i[:], w1h[:], w2i[:], w2h[:]
    b0i_, b0h_ = b0i[:], b0h[:]
    b1i_, b1h_ = b1i[:], b1h[:]
    b2i_, b2h_ = b2i[:], b2h[:]

    npre = (MAX_LEN - 1) * N_GRAPHS
    oh = (sm2_ref[:] == lax.broadcasted_iota(jnp.int32, (npre, VOC), 1)
          ).astype(bf)
    gi0t_ref[:] = jnp.dot(oh, w0i[:], preferred_element_type=_F32)

    def gates(gi, gh, hv):
        r = jax.nn.sigmoid(gi[:, :H] + gh[:, :H])
        zt = jax.nn.sigmoid(gi[:, H:2 * H] + gh[:, H:2 * H])
        n = jnp.tanh(gi[:, 2 * H:] + r * gh[:, 2 * H:])
        return (1.0 - zt) * n + zt * hv

    def cells(gi0, h1, h2, h3):
        gh1 = jnp.dot(h1.astype(bf), w0h_, preferred_element_type=_F32) + b0h_
        gh2 = jnp.dot(h2.astype(bf), w1h_, preferred_element_type=_F32) + b1h_
        gh3 = jnp.dot(h3.astype(bf), w2h_, preferred_element_type=_F32) + b2h_
        h1n = gates(gi0, gh1, h1)
        gi1 = jnp.dot(h1n.astype(bf), w1i_, preferred_element_type=_F32) + b1i_
        h2n = gates(gi1, gh2, h2)
        gi2 = jnp.dot(h2n.astype(bf), w2i_, preferred_element_type=_F32) + b2i_
        h3n = gates(gi2, gh3, h3)
        return h1n, h2n, h3n

    gi0 = jnp.dot(r0_ref[:].astype(bf), w0i[:], preferred_element_type=_F32) + b0i_
    h1, h2, h3 = cells(gi0, h1_ref[:], h2_ref[:], h3_ref[:])
    h3s_ref[pl.ds(0, 1)] = h3[None]

    def substep(t, carry):
        c1, c2, c3 = carry
        g = gi0t_ref[pl.ds((t - 1) * N_GRAPHS, N_GRAPHS)] + b0i_
        c1, c2, c3 = cells(g, c1, c2, c3)
        h3s_ref[pl.ds(t, 1)] = c3[None]
        return (c1, c2, c3)

    def step2(k, carry):
        carry = substep(2 * k + 1, carry)
        return substep(2 * k + 2, carry)

    lax.fori_loop(0, (MAX_LEN - 1) // 2, step2, (h1, h2, h3))
    h3all = h3s_ref[:].reshape(MAX_LEN * N_GRAPHS, H)
    out_ref[:] = jnp.dot(h3all.astype(bf), lw[:], preferred_element_type=_F32) + lb[:]


def _gru(smiles, rnn0, hdec0, pp):
    sm2 = smiles[:, :MAX_LEN - 1].T.reshape((MAX_LEN - 1) * N_GRAPHS, 1)
    bf = jnp.bfloat16
    args = [sm2, rnn0, hdec0[0], hdec0[1], hdec0[2]]
    for l in range(3):
        args.append(pp['gru%d_Wih' % l].T.astype(bf))
        args.append(pp['gru%d_Whh' % l].T.astype(bf))
        args.append(pp['gru%d_bih' % l].reshape(1, -1))
        args.append(pp['gru%d_bhh' % l].reshape(1, -1))
    args.append(pp['lin_W'].astype(bf))
    args.append(pp['lin_b'].reshape(1, -1))
    return pl.pallas_call(
        _gru_body,
        out_shape=jax.ShapeDtypeStruct((MAX_LEN * N_GRAPHS, VOC), _F32),
        scratch_shapes=[
            pltpu.VMEM(((MAX_LEN - 1) * N_GRAPHS, 3 * H), _F32),
            pltpu.VMEM((MAX_LEN, N_GRAPHS, H), _F32),
        ],
    )(*args)


_NC, _NS = 2, 16
_NW = _NC * _NS
_CH = 128
_NROWS = 2560
_AGG = 10240
_RPT = _AGG // _NS
_DUMMY = 10000
_ZR = 40
_SB = 16
_RPW = 80


def _sc_gather_scatter(table, fidx2d, dst2d):
    mesh = plsc.VectorSubcoreMesh(core_axis_name="c", subcore_axis_name="s")

    def _body(cid, sid, table_hbm, fidx_hbm, dst_hbm, out_hbm, fblk, dblk,
              rows_v, zbuf_v, agg_sh, sem_g, sem_s, sem_i):
        zv = jnp.zeros((16,), _F32)

        def zstore(j, _):
            zbuf_v[j // 8, pl.ds((j % 8) * 16, 16)] = zv
            return 0

        lax.fori_loop(0, _ZR * 8, zstore, 0)
        for t in range(_RPT // _ZR):
            pltpu.sync_copy(zbuf_v, agg_sh.at[pl.ds(sid * _RPT + t * _ZR, _ZR)])
        plsc.subcore_barrier()

        base = (sid * _NC + cid) * _RPW
        nsb = _RPW // _SB
        pltpu.sync_copy(fidx_hbm.at[pl.ds(base, _SB)], fblk.at[0])
        pltpu.sync_copy(dst_hbm.at[pl.ds(base, _SB)], dblk.at[0])

        def superblock(s, _):
            sb = s % 2
            nb = (s + 1) % 2
            nxt = base + (s + 1) * _SB

            @pl.when(s < nsb - 1)
            def _pf():
                pltpu.async_copy(fidx_hbm.at[pl.ds(nxt, _SB)], fblk.at[nb], sem_i)
                pltpu.async_copy(dst_hbm.at[pl.ds(nxt, _SB)], dblk.at[nb], sem_i)

            pending = [None, None]
            pend_g = [None, None]
            pend_g[0] = pltpu.async_copy(table_hbm.at[fblk.at[sb, 0]],
                                         rows_v.at[0], sem_g[0])
            for j in range(_SB):
                b = j % 2
                nb2 = 1 - b
                if j + 1 < _SB:
                    if pending[nb2] is not None:
                        pending[nb2].wait()
                        pending[nb2] = None
                    pend_g[nb2] = pltpu.async_copy(
                        table_hbm.at[fblk.at[sb, j + 1]], rows_v.at[nb2],
                        sem_g[nb2])
                pend_g[b].wait()
                pending[b] = pltpu.async_copy(rows_v.at[b],
                                              agg_sh.at[dblk.at[sb, j]],
                                              sem_s[b], add=True)
            pending[0].wait()
            pending[1].wait()

            @pl.when(s < nsb - 1)
            def _pfw():
                pltpu.make_async_copy(fidx_hbm.at[pl.ds(nxt, _SB)], fblk.at[nb],
                                      sem_i).wait()
                pltpu.make_async_copy(dst_hbm.at[pl.ds(nxt, _SB)], dblk.at[nb],
                                      sem_i).wait()

            return 0

        lax.fori_loop(0, nsb, superblock, 0)

        plsc.subcore_barrier()
        rbase = sid * _RPT

        @pl.when(sid < _NS - 1)
        def _full():
            pltpu.sync_copy(agg_sh.at[pl.ds(rbase, _RPT)],
                            out_hbm.at[cid, pl.ds(rbase, _RPT)])

        @pl.when(sid == _NS - 1)
        def _last():
            lastbase = (_NS - 1) * _RPT
            pltpu.sync_copy(agg_sh.at[pl.ds(lastbase, N_NODES - lastbase)],
                            out_hbm.at[cid, pl.ds(lastbase, N_NODES - lastbase)])

    @functools.partial(
        pl.kernel, mesh=mesh,
        out_type=jax.ShapeDtypeStruct((_NC, N_NODES, D), _F32),
        scratch_types=[
            pltpu.VMEM((2, _SB, _CH), jnp.int32),
            pltpu.VMEM((2, _SB, _CH), jnp.int32),
            pltpu.VMEM((2, _CH, D), _F32),
            pltpu.VMEM((_ZR, D), _F32),
            pltpu.VMEM_SHARED((_AGG, D), _F32),
            pltpu.SemaphoreType.DMA,
            pltpu.SemaphoreType.DMA,
            pltpu.SemaphoreType.DMA,
            pltpu.SemaphoreType.DMA,
            pltpu.SemaphoreType.DMA,
        ],
    )
    def k(table_hbm, fidx_hbm, dst_hbm, out_hbm, fblk, dblk, rows_v, zbuf_v,
          agg_sh, sem_g0, sem_g1, sem_s0, sem_s1, sem_i):
        cid = lax.axis_index("c")
        sid = lax.axis_index("s")

        _body(cid, sid, table_hbm, fidx_hbm, dst_hbm, out_hbm, fblk, dblk,
              rows_v, zbuf_v, agg_sh, (sem_g0, sem_g1), (sem_s0, sem_s1),
              sem_i)

    return k(table, fidx2d, dst2d)


def _build_idx_body(et_ref, src_ref, dst_ref, f_ref, d_ref):
    nreal = N_EDGES // _CH
    npadr = _NROWS - nreal
    f_ref[pl.ds(0, nreal)] = et_ref[:] * N_NODES + src_ref[:]
    d_ref[pl.ds(0, nreal)] = dst_ref[:]
    flat = (lax.broadcasted_iota(jnp.int32, (npadr, _CH), 0) * _CH
            + lax.broadcasted_iota(jnp.int32, (npadr, _CH), 1))
    f_ref[pl.ds(nreal, npadr)] = flat % (NUM_RELS * N_NODES)
    d_ref[pl.ds(nreal, npadr)] = _DUMMY + flat % (_AGG - _DUMMY)


def _build_idx(et2d, src2d, dst2d):
    return pl.pallas_call(
        _build_idx_body,
        out_shape=(
            jax.ShapeDtypeStruct((_NROWS, _CH), jnp.int32),
            jax.ShapeDtypeStruct((_NROWS, _CH), jnp.int32),
        ),
    )(et2d, src2d, dst2d)


def kernel(x, edge_index, etypes, graph_ids, smiles, params, eps):
    src = edge_index[0]
    dst = edge_index[1]
    fidx2d, dst2d = _build_idx(etypes.reshape(-1, _CH), src.reshape(-1, _CH),
                               dst.reshape(-1, _CH))

    xw, sterm = _dense0(x, params['rgcn0_W'], params['rgcn0_Wself'], params['rgcn0_b'])
    parts = _sc_gather_scatter(xw, fidx2d, dst2d)
    for l in (1, 2, 3):
        xw, sterm = _densef(parts, sterm, params['rgcn%d_W' % l],
                            params['rgcn%d_Wself' % l], params['rgcn%d_b' % l])
        parts = _sc_gather_scatter(xw, fidx2d, dst2d)

    pooled = _pool(parts, sterm, graph_ids)
    mu2, lv2, z, rnn0, dinit, props, affs = _encoder(pooled, eps.reshape(N_GRAPHS, L_SIZE), params)
    hdec0 = dinit.reshape(3, N_GRAPHS, H)
    outs = _gru(smiles, rnn0, hdec0, params).reshape(MAX_LEN, N_GRAPHS, VOC)
    gen_seq = jnp.transpose(outs, (1, 2, 0))
    mu = mu2.reshape(N_GRAPHS, 1, L_SIZE)
    logv = lv2.reshape(N_GRAPHS, 1, L_SIZE)
    return mu, logv, z, gen_seq, props, affs
